# baseline (speedup 1.0000x reference)
import functools

import jax
import jax.numpy as jnp
from jax.experimental import pallas as pl
from jax.experimental.pallas import tpu as pltpu

B, S, C, D, TOPN = 16, 32, 4096, 256, 8


def _topk_kernel(q_ref, ctx_ref, w_ref, b_ref, dist_out_ref, idx_out_ref):
    q = q_ref[0]
    w = w_ref[...]
    bias = b_ref[...]
    qp = jax.lax.dot_general(q, w, (((1,), (1,)), ((), ()))) + bias

    ctx = ctx_ref[0]
    a2 = jnp.sum(qp * qp, axis=-1, keepdims=True)
    b2 = jnp.sum(ctx * ctx, axis=-1)
    ab = jax.lax.dot_general(qp, ctx, (((1,), (1,)), ((), ())))
    d2 = jnp.maximum(a2 + b2[None, :] - 2.0 * ab, 0.0)
    dist = jnp.sqrt(d2)

    nch = 4
    w = C // nch
    inf = jnp.float32(jnp.inf)
    cand_v = [[] for _ in range(nch)]
    cand_i = [[] for _ in range(nch)]
    chunk_vals = []
    chunk_iota = []
    for c in range(nch):
        chunk_vals.append(dist[:, c * w:(c + 1) * w])
        chunk_iota.append(
            jax.lax.broadcasted_iota(jnp.int32, (S, w), 1) + jnp.int32(c * w))
    for _ in range(TOPN):
        for c in range(nch):
            vals = chunk_vals[c]
            iota = chunk_iota[c]
            mv = jnp.min(vals, axis=1, keepdims=True)
            eq = vals == mv
            mi = jnp.min(jnp.where(eq, iota, C), axis=1, keepdims=True)
            cand_v[c].append(mv)
            cand_i[c].append(mi)
            chunk_vals[c] = jnp.where(iota == mi, inf, vals)
    mrg_v = jnp.concatenate([v for vs in cand_v for v in vs], axis=1)
    mrg_i = jnp.concatenate([i for is_ in cand_i for i in is_], axis=1)
    top_vals = []
    top_idx = []
    for _ in range(TOPN):
        mv = jnp.min(mrg_v, axis=1, keepdims=True)
        eq = mrg_v == mv
        mi = jnp.min(jnp.where(eq, mrg_i, C), axis=1, keepdims=True)
        top_vals.append(mv)
        top_idx.append(mi)
        sel = eq & (mrg_i == mi)
        mrg_v = jnp.where(sel, inf, mrg_v)
    dist_out_ref[0] = jnp.concatenate(top_vals, axis=1)
    idx_out_ref[0] = jnp.concatenate(top_idx, axis=1)


@jax.jit
def kernel(query_embeddings, context_embeddings, W, b):
    bias2d = b.reshape(1, D)
    grid = (B,)
    out_dist, out_idx = pl.pallas_call(
        _topk_kernel,
        grid=grid,
        in_specs=[
            pl.BlockSpec((1, S, D), lambda i: (i, 0, 0)),
            pl.BlockSpec((1, C, D), lambda i: (i, 0, 0)),
            pl.BlockSpec((D, D), lambda i: (0, 0)),
            pl.BlockSpec((1, D), lambda i: (0, 0)),
        ],
        out_specs=[
            pl.BlockSpec((1, S, TOPN), lambda i: (i, 0, 0)),
            pl.BlockSpec((1, S, TOPN), lambda i: (i, 0, 0)),
        ],
        out_shape=[
            jax.ShapeDtypeStruct((B, S, TOPN), jnp.float32),
            jax.ShapeDtypeStruct((B, S, TOPN), jnp.int32),
        ],
        compiler_params=pltpu.CompilerParams(
            dimension_semantics=("parallel",),
        ),
    )(query_embeddings, context_embeddings, W, bias2d)
    return (out_dist, out_idx)


# speedup vs baseline: 1.6693x; 1.6693x over previous
import functools

import jax
import jax.numpy as jnp
from jax.experimental import pallas as pl
from jax.experimental.pallas import tpu as pltpu

B, S, C, D, TOPN = 16, 32, 4096, 256, 8


def _topk_kernel(q_ref, ctx_ref, w_ref, b_ref, dist_out_ref, idx_out_ref):
    q = q_ref[0]
    w = w_ref[...]
    bias = b_ref[...]
    qp = jax.lax.dot_general(q, w, (((1,), (1,)), ((), ()))) + bias

    ctx = ctx_ref[0]
    a2 = jnp.sum(qp * qp, axis=-1, keepdims=True)
    b2 = jnp.sum(ctx * ctx, axis=-1)
    ab = jax.lax.dot_general(qp, ctx, (((1,), (1,)), ((), ())))
    d2 = jnp.maximum(a2 + b2[None, :] - 2.0 * ab, 0.0)
    dist = jnp.sqrt(d2)

    iota = jax.lax.broadcasted_iota(jnp.int32, (S, C), 1)
    vals = dist
    top_vals = []
    top_idx = []
    for _ in range(TOPN):
        mv = jnp.min(vals, axis=1, keepdims=True)
        mi = jnp.argmin(vals, axis=1, keepdims=True).astype(jnp.int32)
        top_vals.append(mv)
        top_idx.append(mi)
        vals = jnp.where(iota == mi, jnp.float32(jnp.inf), vals)
    dist_out_ref[0] = jnp.concatenate(top_vals, axis=1)
    idx_out_ref[0] = jnp.concatenate(top_idx, axis=1)


@jax.jit
def kernel(query_embeddings, context_embeddings, W, b):
    bias2d = b.reshape(1, D)
    grid = (B,)
    out_dist, out_idx = pl.pallas_call(
        _topk_kernel,
        grid=grid,
        in_specs=[
            pl.BlockSpec((1, S, D), lambda i: (i, 0, 0)),
            pl.BlockSpec((1, C, D), lambda i: (i, 0, 0)),
            pl.BlockSpec((D, D), lambda i: (0, 0)),
            pl.BlockSpec((1, D), lambda i: (0, 0)),
        ],
        out_specs=[
            pl.BlockSpec((1, S, TOPN), lambda i: (i, 0, 0)),
            pl.BlockSpec((1, S, TOPN), lambda i: (i, 0, 0)),
        ],
        out_shape=[
            jax.ShapeDtypeStruct((B, S, TOPN), jnp.float32),
            jax.ShapeDtypeStruct((B, S, TOPN), jnp.int32),
        ],
        compiler_params=pltpu.CompilerParams(
            dimension_semantics=("parallel",),
        ),
    )(query_embeddings, context_embeddings, W, bias2d)
    return (out_dist, out_idx)


# speedup vs baseline: 1.9664x; 1.1779x over previous
import functools

import jax
import jax.numpy as jnp
from jax.experimental import pallas as pl
from jax.experimental.pallas import tpu as pltpu

B, S, C, D, TOPN = 16, 32, 4096, 256, 8
BB = 2


def _one_batch(q, ctx, w, bias, dist_out_ref, idx_out_ref, j):
    qp = jax.lax.dot_general(q, w, (((1,), (1,)), ((), ()))) + bias

    a2 = jnp.sum(qp * qp, axis=-1, keepdims=True)
    b2 = jnp.sum(ctx * ctx, axis=-1)
    ab = jax.lax.dot_general(qp, ctx, (((1,), (1,)), ((), ())))
    d2 = jnp.maximum(a2 + b2[None, :] - 2.0 * ab, 0.0)
    dist = jnp.sqrt(d2)

    iota = jax.lax.broadcasted_iota(jnp.int32, (S, C), 1)
    vals = dist
    top_vals = []
    top_idx = []
    for _ in range(TOPN):
        mv = jnp.min(vals, axis=1, keepdims=True)
        mi = jnp.argmin(vals, axis=1, keepdims=True).astype(jnp.int32)
        top_vals.append(mv)
        top_idx.append(mi)
        vals = jnp.where(iota == mi, jnp.float32(jnp.inf), vals)
    dist_out_ref[j] = jnp.concatenate(top_vals, axis=1)
    idx_out_ref[j] = jnp.concatenate(top_idx, axis=1)


def _topk_kernel(q_ref, ctx_ref, w_ref, b_ref, dist_out_ref, idx_out_ref):
    w = w_ref[...]
    bias = b_ref[...]
    for j in range(BB):
        _one_batch(q_ref[j], ctx_ref[j], w, bias, dist_out_ref, idx_out_ref, j)


@jax.jit
def kernel(query_embeddings, context_embeddings, W, b):
    bias2d = b.reshape(1, D)
    grid = (B // BB,)
    out_dist, out_idx = pl.pallas_call(
        _topk_kernel,
        grid=grid,
        in_specs=[
            pl.BlockSpec((BB, S, D), lambda i: (i, 0, 0)),
            pl.BlockSpec((BB, C, D), lambda i: (i, 0, 0)),
            pl.BlockSpec((D, D), lambda i: (0, 0)),
            pl.BlockSpec((1, D), lambda i: (0, 0)),
        ],
        out_specs=[
            pl.BlockSpec((BB, S, TOPN), lambda i: (i, 0, 0)),
            pl.BlockSpec((BB, S, TOPN), lambda i: (i, 0, 0)),
        ],
        out_shape=[
            jax.ShapeDtypeStruct((B, S, TOPN), jnp.float32),
            jax.ShapeDtypeStruct((B, S, TOPN), jnp.int32),
        ],
        compiler_params=pltpu.CompilerParams(
            dimension_semantics=("parallel",),
        ),
    )(query_embeddings, context_embeddings, W, bias2d)
    return (out_dist, out_idx)
